# baseline (speedup 1.0000x reference)
import functools

import jax
import jax.numpy as jnp
from jax import lax
from jax.experimental import pallas as pl
from jax.experimental.pallas import tpu as pltpu
from jax.experimental.pallas import tpu_sc as plsc


_TOK_BLK = 1024
_CODE_CHUNK = 1024


def _argmin_body(z_ref, cb_ref, idx_ref, loss_ref, run_m_ref, run_a_ref):
    i = pl.program_id(0)
    j = pl.program_id(1)
    n_chunks = pl.num_programs(1)
    zb = z_ref[...]
    ct = cb_ref[...]
    cn = jnp.sum(ct * ct, axis=0, keepdims=True)
    prod = lax.dot_general(
        zb, ct,
        (((1,), (0,)), ((), ())),
        preferred_element_type=jnp.float32)
    zn = jnp.sum(zb * zb, axis=1, keepdims=True)
    dist = (zn - 2.0 * prod) + cn
    m = jnp.min(dist, axis=1, keepdims=True)
    lanes = lax.broadcasted_iota(jnp.int32, (_TOK_BLK, _CODE_CHUNK), 1)
    a = jnp.min(jnp.where(dist == m, lanes, jnp.int32(2**30)),
                axis=1, keepdims=True) + j * _CODE_CHUNK

    @pl.when(j == 0)
    def _():
        run_m_ref[...] = m
        run_a_ref[...] = a

    @pl.when(j > 0)
    def _():
        take = m < run_m_ref[...]
        run_m_ref[...] = jnp.where(take, m, run_m_ref[...])
        run_a_ref[...] = jnp.where(take, a, run_a_ref[...])

    @pl.when(j == n_chunks - 1)
    def _():
        idx_ref[0, 0, :] = run_a_ref[...][:, 0]
        block_loss = jnp.sum(run_m_ref[...])

        @pl.when(i == 0)
        def _():
            loss_ref[0, 0] = block_loss

        @pl.when(i > 0)
        def _():
            loss_ref[0, 0] = loss_ref[0, 0] + block_loss


def _argmin_call(flat, cb_t):
    n, d = flat.shape
    v = cb_t.shape[1]
    nb = n // _TOK_BLK
    n_chunks = v // _CODE_CHUNK
    return pl.pallas_call(
        _argmin_body,
        grid=(nb, n_chunks),
        in_specs=[
            pl.BlockSpec((_TOK_BLK, d), lambda i, j: (i, 0)),
            pl.BlockSpec((d, _CODE_CHUNK), lambda i, j: (0, j)),
        ],
        out_specs=[
            pl.BlockSpec((1, 1, _TOK_BLK), lambda i, j: (i, 0, 0)),
            pl.BlockSpec(block_shape=(1, 1), index_map=lambda i, j: (0, 0),
                         memory_space=pltpu.SMEM),
        ],
        out_shape=[
            jax.ShapeDtypeStruct((nb, 1, _TOK_BLK), jnp.int32),
            jax.ShapeDtypeStruct((1, 1), jnp.float32),
        ],
        scratch_shapes=[
            pltpu.VMEM((_TOK_BLK, 1), jnp.float32),
            pltpu.VMEM((_TOK_BLK, 1), jnp.int32),
        ],
    )(flat, cb_t)


def _sc_gather(indices, codebook):
    n = indices.shape[0]
    d = codebook.shape[1]
    info = plsc.get_sparse_core_info()
    nw = info.num_cores * info.num_subcores
    b_per_w = n // nw
    mesh = plsc.VectorSubcoreMesh(core_axis_name="c", subcore_axis_name="s")

    @functools.partial(
        pl.kernel, mesh=mesh,
        compiler_params=pltpu.CompilerParams(use_tc_tiling_on_sc=False),
        out_type=jax.ShapeDtypeStruct((n, d), jnp.float32),
        scratch_types=[
            pltpu.VMEM((b_per_w,), jnp.int32),
            pltpu.VMEM((b_per_w, d), jnp.float32),
            pltpu.SemaphoreType.DMA,
        ],
    )
    def gather_k(idx_hbm, table_hbm, out_hbm, idx_v, rows_v, sem):
        wid = lax.axis_index("s") * info.num_cores + lax.axis_index("c")
        base = wid * b_per_w
        pltpu.sync_copy(idx_hbm.at[pl.ds(base, b_per_w)], idx_v)
        pltpu.async_copy(table_hbm.at[idx_v], rows_v, sem).wait()
        pltpu.sync_copy(rows_v, out_hbm.at[pl.ds(base, b_per_w)])

    return gather_k(indices, codebook)


def kernel(z, codebook):
    b, n, d = z.shape
    flat = z.reshape(-1, d)
    idx_blocks, loss_sum = _argmin_call(flat, codebook.T)
    indices = idx_blocks.reshape(-1)
    quantized = _sc_gather(indices, codebook).reshape(b, n, d)
    quantize_st = z + lax.stop_gradient(quantized - z)
    commit_loss = loss_sum[0, 0] / jnp.float32(flat.shape[0] * d)
    indices = indices.reshape(b, n)
    return quantize_st, indices, commit_loss
